# initial kernel scaffold (unmeasured)
import functools

import jax
import jax.numpy as jnp
from jax import lax
from jax.experimental import pallas as pl
from jax.experimental.pallas import tpu as pltpu

N_DEV = 4
KV_CHUNK = 512


def _flash_partial_body(
    q_ref, k_ref, v_ref, o_ref, m_ref, l_ref, acc_s, m_s, l_s, *, num_chunks, scale
):
    ci = pl.program_id(1)

    @pl.when(ci == 0)
    def _():
        m_s[...] = jnp.full(m_s.shape, -jnp.inf, jnp.float32)
        l_s[...] = jnp.zeros(l_s.shape, jnp.float32)
        acc_s[...] = jnp.zeros(acc_s.shape, jnp.float32)

    q = q_ref[0].astype(jnp.bfloat16)
    k = k_ref[0].astype(jnp.bfloat16)
    v = v_ref[0].astype(jnp.bfloat16)

    s = lax.dot_general(
        q, k, (((2,), (2,)), ((1,), (1,))), preferred_element_type=jnp.float32
    )
    s = s * scale
    m_c = jnp.max(s, axis=-1)
    m_prev = m_s[...]
    m_new = jnp.maximum(m_prev, m_c)
    alpha = jnp.exp(m_prev - m_new)
    p = jnp.exp(s - m_new[:, :, None])
    pv = lax.dot_general(
        p.astype(jnp.bfloat16),
        v,
        (((2,), (0,)), ((0,), (1,))),
        preferred_element_type=jnp.float32,
    )
    acc_s[...] = acc_s[...] * alpha[:, :, None] + pv
    l_s[...] = l_s[...] * alpha + jnp.sum(p, axis=-1)
    m_s[...] = m_new

    @pl.when(ci == num_chunks - 1)
    def _():
        o_ref[0] = acc_s[...]
        m_ref[0] = m_s[...]
        l_ref[0] = l_s[...]


def _flash_partial(Q, K, V):
    B, Sq, H, D = Q.shape
    Skv = K.shape[1]
    num_chunks = Skv // KV_CHUNK
    scale = D**-0.5
    body = functools.partial(_flash_partial_body, num_chunks=num_chunks, scale=scale)
    return pl.pallas_call(
        body,
        grid=(B, num_chunks),
        in_specs=[
            pl.BlockSpec((1, Sq, H, D), lambda b, c: (b, 0, 0, 0)),
            pl.BlockSpec((1, KV_CHUNK, H, D), lambda b, c: (b, c, 0, 0)),
            pl.BlockSpec((1, KV_CHUNK, H, D), lambda b, c: (b, c, 0, 0)),
        ],
        out_specs=[
            pl.BlockSpec((1, H, Sq, D), lambda b, c: (b, 0, 0, 0)),
            pl.BlockSpec((1, H, Sq), lambda b, c: (b, 0, 0)),
            pl.BlockSpec((1, H, Sq), lambda b, c: (b, 0, 0)),
        ],
        out_shape=[
            jax.ShapeDtypeStruct((B, H, Sq, D), jnp.float32),
            jax.ShapeDtypeStruct((B, H, Sq), jnp.float32),
            jax.ShapeDtypeStruct((B, H, Sq), jnp.float32),
        ],
        scratch_shapes=[
            pltpu.VMEM((H, Sq, D), jnp.float32),
            pltpu.VMEM((H, Sq), jnp.float32),
            pltpu.VMEM((H, Sq), jnp.float32),
        ],
        compiler_params=pltpu.CompilerParams(
            dimension_semantics=("arbitrary", "arbitrary"),
        ),
    )(Q, K, V)


def _allreduce_body(
    o_ref, m_ref, l_ref, out_ref, obuf, mbuf, lbuf, co, cm, cl, send_sems, recv_sems
):
    my = lax.axis_index("i")
    p1 = jnp.bitwise_xor(my, 1)
    p2 = 3 - my

    barrier = pltpu.get_barrier_semaphore()
    for nbr in (p1, p2):
        pl.semaphore_signal(
            barrier, inc=1, device_id=(nbr,), device_id_type=pl.DeviceIdType.MESH
        )
    pl.semaphore_wait(barrier, 2)

    def exchange(r, partner, src_o, src_m, src_l):
        copies = []
        for j, (src, dst) in enumerate(
            ((src_o, obuf), (src_m, mbuf), (src_l, lbuf))
        ):
            rdma = pltpu.make_async_remote_copy(
                src_ref=src,
                dst_ref=dst.at[r],
                send_sem=send_sems.at[r, j],
                recv_sem=recv_sems.at[r, j],
                device_id=(partner,),
                device_id_type=pl.DeviceIdType.MESH,
            )
            rdma.start()
            copies.append(rdma)
        for rdma in copies:
            rdma.wait()

    def combine(r, src_o, src_m, src_l):
        m_a = src_m[...]
        m_b = mbuf[r]
        m_new = jnp.maximum(m_a, m_b)
        w_a = jnp.exp(m_a - m_new)
        w_b = jnp.exp(m_b - m_new)
        l_new = src_l[...] * w_a + lbuf[r] * w_b
        o_new = src_o[...] * w_a[..., None] + obuf[r] * w_b[..., None]
        return o_new, m_new, l_new

    exchange(0, p1, o_ref, m_ref, l_ref)
    o1, m1, l1 = combine(0, o_ref, m_ref, l_ref)
    co[...] = o1
    cm[...] = m1
    cl[...] = l1
    exchange(1, p2, co, cm, cl)
    o2, _, l2 = combine(1, co, cm, cl)
    out = o2 / l2[..., None]
    out_ref[...] = jnp.transpose(out, (0, 2, 1, 3))


def _combine_allreduce(o, m, l):
    B, H, Sq, D = o.shape
    return pl.pallas_call(
        _allreduce_body,
        out_shape=jax.ShapeDtypeStruct((B, Sq, H, D), jnp.float32),
        in_specs=[pl.BlockSpec(memory_space=pltpu.VMEM)] * 3,
        out_specs=pl.BlockSpec(memory_space=pltpu.VMEM),
        scratch_shapes=[
            pltpu.VMEM((2, B, H, Sq, D), jnp.float32),
            pltpu.VMEM((2, B, H, Sq), jnp.float32),
            pltpu.VMEM((2, B, H, Sq), jnp.float32),
            pltpu.VMEM((B, H, Sq, D), jnp.float32),
            pltpu.VMEM((B, H, Sq), jnp.float32),
            pltpu.VMEM((B, H, Sq), jnp.float32),
            pltpu.SemaphoreType.DMA((2, 3)),
            pltpu.SemaphoreType.DMA((2, 3)),
        ],
        compiler_params=pltpu.CompilerParams(collective_id=0),
    )(o, m, l)


def kernel(Q, K, V):
    o, m, l = _flash_partial(Q, K, V)
    return _combine_allreduce(o, m, l)


# baseline (device time: 197177 ns/iter reference)
import functools

import jax
import jax.numpy as jnp
from jax import lax
from jax.experimental import pallas as pl
from jax.experimental.pallas import tpu as pltpu

N_DEV = 4
KV_CHUNK = 256


def _flash_partial_body(
    q_ref, k_ref, v_ref, o_ref, m_ref, l_ref, acc_s, m_s, l_s, *, num_chunks, scale
):
    ci = pl.program_id(1)

    @pl.when(ci == 0)
    def _():
        m_s[...] = jnp.full(m_s.shape, -jnp.inf, jnp.float32)
        l_s[...] = jnp.zeros(l_s.shape, jnp.float32)
        acc_s[...] = jnp.zeros(acc_s.shape, jnp.float32)

    q = q_ref[0].astype(jnp.bfloat16)
    k = k_ref[0].astype(jnp.bfloat16)
    v = v_ref[0].astype(jnp.bfloat16)

    s = lax.dot_general(
        q, k, (((2,), (2,)), ((1,), (1,))), preferred_element_type=jnp.float32
    )
    s = s * scale
    m_c = jnp.max(s, axis=-1)
    m_prev = m_s[...]
    m_new = jnp.maximum(m_prev, m_c)
    alpha = jnp.exp(m_prev - m_new)
    p = jnp.exp(s - m_new[:, :, None])
    pv = lax.dot_general(
        p.astype(jnp.bfloat16),
        v,
        (((2,), (0,)), ((0,), (1,))),
        preferred_element_type=jnp.float32,
    )
    acc_s[...] = acc_s[...] * alpha[:, :, None] + pv
    l_s[...] = l_s[...] * alpha + jnp.sum(p, axis=-1)
    m_s[...] = m_new

    @pl.when(ci == num_chunks - 1)
    def _():
        o_ref[0] = acc_s[...]
        m_ref[0] = m_s[...]
        l_ref[0] = l_s[...]


def _flash_partial(Q, K, V):
    B, Sq, H, D = Q.shape
    Skv = K.shape[1]
    num_chunks = Skv // KV_CHUNK
    scale = D**-0.5
    body = functools.partial(_flash_partial_body, num_chunks=num_chunks, scale=scale)
    return pl.pallas_call(
        body,
        grid=(B, num_chunks),
        in_specs=[
            pl.BlockSpec((1, Sq, H, D), lambda b, c: (b, 0, 0, 0)),
            pl.BlockSpec((1, KV_CHUNK, H, D), lambda b, c: (b, c, 0, 0)),
            pl.BlockSpec((1, KV_CHUNK, H, D), lambda b, c: (b, c, 0, 0)),
        ],
        out_specs=[
            pl.BlockSpec((1, H, Sq, D), lambda b, c: (b, 0, 0, 0)),
            pl.BlockSpec((1, H, Sq), lambda b, c: (b, 0, 0)),
            pl.BlockSpec((1, H, Sq), lambda b, c: (b, 0, 0)),
        ],
        out_shape=[
            jax.ShapeDtypeStruct((B, H, Sq, D), jnp.float32),
            jax.ShapeDtypeStruct((B, H, Sq), jnp.float32),
            jax.ShapeDtypeStruct((B, H, Sq), jnp.float32),
        ],
        scratch_shapes=[
            pltpu.VMEM((H, Sq, D), jnp.float32),
            pltpu.VMEM((H, Sq), jnp.float32),
            pltpu.VMEM((H, Sq), jnp.float32),
        ],
        compiler_params=pltpu.CompilerParams(
            dimension_semantics=("arbitrary", "arbitrary"),
        ),
    )(Q, K, V)


def _allreduce_body(
    o_ref, m_ref, l_ref, out_ref, obuf, mbuf, lbuf, co, cm, cl, send_sems, recv_sems
):
    my = lax.axis_index("i")
    p1 = jnp.bitwise_xor(my, 1)
    p2 = 3 - my

    barrier = pltpu.get_barrier_semaphore()
    for nbr in (p1, p2):
        pl.semaphore_signal(
            barrier, inc=1, device_id=(nbr,), device_id_type=pl.DeviceIdType.MESH
        )
    pl.semaphore_wait(barrier, 2)

    def exchange(r, partner, src_o, src_m, src_l):
        copies = []
        for j, (src, dst) in enumerate(
            ((src_o, obuf), (src_m, mbuf), (src_l, lbuf))
        ):
            rdma = pltpu.make_async_remote_copy(
                src_ref=src,
                dst_ref=dst.at[r],
                send_sem=send_sems.at[r, j],
                recv_sem=recv_sems.at[r, j],
                device_id=(partner,),
                device_id_type=pl.DeviceIdType.MESH,
            )
            rdma.start()
            copies.append(rdma)
        for rdma in copies:
            rdma.wait()

    def combine(r, src_o, src_m, src_l):
        m_a = src_m[...]
        m_b = mbuf[r]
        m_new = jnp.maximum(m_a, m_b)
        w_a = jnp.exp(m_a - m_new)
        w_b = jnp.exp(m_b - m_new)
        l_new = src_l[...] * w_a + lbuf[r] * w_b
        o_new = src_o[...] * w_a[..., None] + obuf[r] * w_b[..., None]
        return o_new, m_new, l_new

    exchange(0, p1, o_ref, m_ref, l_ref)
    o1, m1, l1 = combine(0, o_ref, m_ref, l_ref)
    co[...] = o1
    cm[...] = m1
    cl[...] = l1
    exchange(1, p2, co, cm, cl)
    o2, _, l2 = combine(1, co, cm, cl)
    out = o2 / l2[..., None]
    out_ref[...] = jnp.transpose(out, (0, 2, 1, 3))


def _combine_allreduce(o, m, l):
    B, H, Sq, D = o.shape
    return pl.pallas_call(
        _allreduce_body,
        out_shape=jax.ShapeDtypeStruct((B, Sq, H, D), jnp.float32),
        in_specs=[pl.BlockSpec(memory_space=pltpu.VMEM)] * 3,
        out_specs=pl.BlockSpec(memory_space=pltpu.VMEM),
        scratch_shapes=[
            pltpu.VMEM((2, B, H, Sq, D), jnp.float32),
            pltpu.VMEM((2, B, H, Sq), jnp.float32),
            pltpu.VMEM((2, B, H, Sq), jnp.float32),
            pltpu.VMEM((B, H, Sq, D), jnp.float32),
            pltpu.VMEM((B, H, Sq), jnp.float32),
            pltpu.VMEM((B, H, Sq), jnp.float32),
            pltpu.SemaphoreType.DMA((2, 3)),
            pltpu.SemaphoreType.DMA((2, 3)),
        ],
        compiler_params=pltpu.CompilerParams(collective_id=0),
    )(o, m, l)


def kernel(Q, K, V):
    o, m, l = _flash_partial(Q, K, V)
    return _combine_allreduce(o, m, l)
